# baseline (speedup 1.0000x reference)
import functools

import jax
import jax.numpy as jnp
from jax import lax
from jax.experimental import pallas as pl
from jax.experimental.pallas import tpu as pltpu
from jax.experimental.pallas import tpu_sc as plsc


@functools.cache
def _make_gather(V, D, F, B):
    assert D == 64 and B % 128 == 0
    NJC = B // 128
    NBLK = F * NJC
    info = plsc.get_sparse_core_info()
    NC, NS = info.num_cores, info.num_subcores
    NW = NC * NS
    assert NBLK % (2 * NW) == 0
    n = NBLK // NW
    mesh = plsc.VectorSubcoreMesh(core_axis_name="c", subcore_axis_name="s")

    @functools.partial(
        pl.kernel,
        mesh=mesh,
        compiler_params=pltpu.CompilerParams(
            use_tc_tiling_on_sc=False, needs_layout_passes=False
        ),
        out_type=jax.ShapeDtypeStruct((F, 8, NJC, 8, 128), jnp.float32),
        scratch_types=[
            pltpu.VMEM((128,), jnp.int32),
            pltpu.VMEM((128,), jnp.int32),
            pltpu.VMEM((128, D), jnp.float32),
            pltpu.VMEM((128, D), jnp.float32),
            pltpu.VMEM((8, 8, 128), jnp.float32),
            pltpu.VMEM((8, 8, 128), jnp.float32),
            pltpu.SemaphoreType.DMA,
            pltpu.SemaphoreType.DMA,
            pltpu.SemaphoreType.DMA,
            pltpu.SemaphoreType.DMA,
            pltpu.SemaphoreType.DMA,
            pltpu.SemaphoreType.DMA,
        ],
    )
    def k(w_hbm, idx_hbm, out_hbm,
          idx0, idx1, rows0, rows1, outb0, outb1,
          semi0, semi1, semg0, semg1, semo0, semo1):
        wid = lax.axis_index("s") * NC + lax.axis_index("c")
        iota16 = lax.iota(jnp.int32, 16)
        idxb = (idx0, idx1)
        rows = (rows0, rows1)
        outb = (outb0, outb1)
        semi = (semi0, semi1)
        semg = (semg0, semg1)
        semo = (semo0, semo1)
        g0 = wid * n

        def cp_idx(g, par):
            off = pl.multiple_of((g0 + g) << 7, 128)
            return pltpu.make_async_copy(
                idx_hbm.at[pl.ds(off, 128)], idxb[par], semi[par]
            )

        def cp_gather(par):
            return pltpu.make_async_copy(
                w_hbm.at[idxb[par]], rows[par], semg[par]
            )

        def cp_out(g, par):
            gg = g0 + g
            f = gg >> 7
            jc = gg & (NJC - 1)
            return pltpu.make_async_copy(
                outb[par], out_hbm.at[f, :, jc, :, :], semo[par]
            )

        def transpose_block(par):
            rbuf = rows[par]
            obuf = outb[par]
            rowvs = [iota16 + 16 * m for m in range(8)]

            def dbody(d, carry):
                colv = jnp.full((16,), d, jnp.int32)
                kk = d >> 3
                s = d & 7
                for m in range(8):
                    vals = plsc.load_gather(rbuf, [rowvs[m], colv])
                    obuf[kk, s, pl.ds(16 * m, 16)] = vals
                return carry

            lax.fori_loop(0, D, dbody, 0)

        cp_idx(0, 0).start()
        cp_idx(1, 1).start()
        cp_idx(0, 0).wait()
        cp_gather(0).start()

        def step(st, carry):
            for par in (0, 1):
                g = 2 * st + par

                @pl.when(g + 1 < n)
                def _():
                    cp_idx(g + 1, 1 - par).wait()
                    cp_gather(1 - par).start()

                    @pl.when(g + 2 < n)
                    def _():
                        cp_idx(g + 2, par).start()

                cp_gather(par).wait()

                @pl.when(g >= 2)
                def _():
                    cp_out(g - 2, par).wait()

                transpose_block(par)
                cp_out(g, par).start()

            return carry

        lax.fori_loop(0, n // 2, step, 0)
        cp_out(n - 2, 0).wait()
        cp_out(n - 1, 1).wait()

    return k


def kernel(weight, indices, padding_idx, scale_grad_by_freq, sparse):
    V, D = weight.shape
    B, F = indices.shape
    flatidx = indices.T.reshape(-1)
    wlin = weight
    out4 = _make_gather(V, D, F, B)(wlin, flatidx)
    return out4.transpose(2, 4, 0, 1, 3).reshape(B, F, D)


# speedup vs baseline: 1.5144x; 1.5144x over previous
import functools

import jax
import jax.numpy as jnp
from jax import lax
from jax.experimental import pallas as pl
from jax.experimental.pallas import tpu as pltpu
from jax.experimental.pallas import tpu_sc as plsc


@functools.cache
def _make_gather(V, D, F, B):
    assert D == 64 and B % 128 == 0
    NJC = B // 128
    NBLK = F * NJC
    info = plsc.get_sparse_core_info()
    NC, NS = info.num_cores, info.num_subcores
    NW = NC * NS
    assert NBLK % (2 * NW) == 0
    n = NBLK // NW
    mesh = plsc.VectorSubcoreMesh(core_axis_name="c", subcore_axis_name="s")

    @functools.partial(
        pl.kernel,
        mesh=mesh,
        compiler_params=pltpu.CompilerParams(
            use_tc_tiling_on_sc=False, needs_layout_passes=False
        ),
        out_type=jax.ShapeDtypeStruct((F, 8, NJC, 8, 128), jnp.float32),
        scratch_types=[
            pltpu.VMEM((128,), jnp.int32),
            pltpu.VMEM((128,), jnp.int32),
            pltpu.VMEM((128, D), jnp.float32),
            pltpu.VMEM((128, D), jnp.float32),
            pltpu.VMEM((8, 8, 129), jnp.float32),
            pltpu.VMEM((8, 8, 129), jnp.float32),
            pltpu.SemaphoreType.DMA,
            pltpu.SemaphoreType.DMA,
            pltpu.SemaphoreType.DMA,
            pltpu.SemaphoreType.DMA,
            pltpu.SemaphoreType.DMA,
            pltpu.SemaphoreType.DMA,
        ],
    )
    def k(w_hbm, idx_hbm, out_hbm,
          idx0, idx1, rows0, rows1, outb0, outb1,
          semi0, semi1, semg0, semg1, semo0, semo1):
        wid = lax.axis_index("s") * NC + lax.axis_index("c")
        iota16 = lax.iota(jnp.int32, 16)
        idxb = (idx0, idx1)
        rows = (rows0, rows1)
        outb = (outb0, outb1)
        semi = (semi0, semi1)
        semg = (semg0, semg1)
        semo = (semo0, semo1)
        g0 = wid * n

        def cp_idx(g, par):
            off = pl.multiple_of((g0 + g) << 7, 128)
            return pltpu.make_async_copy(
                idx_hbm.at[pl.ds(off, 128)], idxb[par], semi[par]
            )

        def cp_gather(par):
            return pltpu.make_async_copy(
                w_hbm.at[idxb[par]], rows[par], semg[par]
            )

        def cp_out(g, par):
            gg = g0 + g
            f = gg >> 7
            jc = gg & (NJC - 1)
            return pltpu.make_async_copy(
                outb[par].at[:, :, pl.ds(0, 128)],
                out_hbm.at[f, :, jc, :, :],
                semo[par],
            )

        dvs = [iota16 + 16 * m for m in range(4)]
        kvs = [dv >> 3 for dv in dvs]
        svs = [dv & 7 for dv in dvs]

        def transpose_block(par):
            rbuf = rows[par]
            obuf = outb[par]

            def tbody(t, carry):
                tv = jnp.full((16,), t, jnp.int32)
                for m in range(4):
                    vals = rbuf[t, pl.ds(16 * m, 16)]
                    plsc.store_scatter(obuf, [kvs[m], svs[m], tv], vals)
                return carry

            lax.fori_loop(0, 128, tbody, 0)

        cp_idx(0, 0).start()
        cp_idx(1, 1).start()
        cp_idx(0, 0).wait()
        cp_gather(0).start()

        def step(st, carry):
            for par in (0, 1):
                g = 2 * st + par

                @pl.when(g + 1 < n)
                def _():
                    cp_idx(g + 1, 1 - par).wait()
                    cp_gather(1 - par).start()

                    @pl.when(g + 2 < n)
                    def _():
                        cp_idx(g + 2, par).start()

                cp_gather(par).wait()

                @pl.when(g >= 2)
                def _():
                    cp_out(g - 2, par).wait()

                transpose_block(par)
                cp_out(g, par).start()

            return carry

        lax.fori_loop(0, n // 2, step, 0)
        cp_out(n - 2, 0).wait()
        cp_out(n - 1, 1).wait()

    return k


def kernel(weight, indices, padding_idx, scale_grad_by_freq, sparse):
    V, D = weight.shape
    B, F = indices.shape
    flatidx = indices.T.reshape(-1)
    wlin = weight
    out4 = _make_gather(V, D, F, B)(wlin, flatidx)
    return out4.transpose(2, 4, 0, 1, 3).reshape(B, F, D)
